# baseline (speedup 1.0000x reference)
import functools

import jax
import jax.numpy as jnp
from jax import lax
from jax.experimental import pallas as pl
from jax.experimental.pallas import tpu as pltpu
from jax.experimental.pallas import tpu_sc as plsc

N = 10000
D = 128
E = 320000
G = 64
C = 16

NC = 2
NS = 16
NT = NC * NS
CH = 128
KCH = 80
IB = 8
KCH0 = 120
KCH1 = 40
NBLK0 = KCH0 // IB
C1ROWS = NS * KCH1
TOTCH = NS * (KCH0 + KCH1)
EPAD = TOTCH * CH
NPAD = 10112
RPT = 624
RPT_LAST = N - RPT * (NS - 1)
ZCH = NPAD // NS // CH

RB = 1000

_mesh = plsc.VectorSubcoreMesh(core_axis_name="c", subcore_axis_name="s")


@functools.partial(
    pl.kernel,
    out_type=jax.ShapeDtypeStruct((NC, N, D), jnp.float32),
    mesh=_mesh,
    scratch_types=[
        pltpu.VMEM((KCH, CH), jnp.int32),
        pltpu.VMEM((CH, D), jnp.float32),
        pltpu.VMEM((CH, D), jnp.float32),
        pltpu.VMEM_SHARED((NPAD, D), jnp.float32),
    ],
)
def _deg_kernel(dst_hbm, out_hbm, didx, ones, zeros, acc):
    c = lax.axis_index("c")
    s = lax.axis_index("s")
    t = c * NS + s

    @pl.loop(0, CH)
    def _(i):
        @pl.loop(0, D, step=16)
        def _(jj):
            ones[i, pl.ds(jj, 16)] = jnp.ones((16,), jnp.float32)
            zeros[i, pl.ds(jj, 16)] = jnp.zeros((16,), jnp.float32)

    @pl.loop(0, NPAD // NS // CH)
    def _(k):
        pltpu.sync_copy(zeros, acc.at[pl.ds(s * (NPAD // NS) + k * CH, CH)])

    pltpu.sync_copy(zeros, acc.at[pl.ds(s * (NPAD // NS) + NPAD // NS - CH, CH)])

    plsc.subcore_barrier()

    pltpu.sync_copy(dst_hbm.at[t], didx)

    @pl.loop(0, KCH)
    def _(j):
        pltpu.sync_copy(ones, acc.at[didx.at[j]], add=True)

    plsc.subcore_barrier()

    @pl.when(s < NS - 1)
    def _():
        pltpu.sync_copy(acc.at[pl.ds(s * RPT, RPT)],
                        out_hbm.at[c, pl.ds(s * RPT, RPT)])

    @pl.when(s == NS - 1)
    def _():
        pltpu.sync_copy(acc.at[pl.ds((NS - 1) * RPT, RPT_LAST)],
                        out_hbm.at[c, pl.ds((NS - 1) * RPT, RPT_LAST)])


@functools.partial(
    pl.kernel,
    out_type=jax.ShapeDtypeStruct((NC, N, D), jnp.float32),
    mesh=_mesh,
    scratch_types=[
        pltpu.VMEM((IB, CH), jnp.int32),
        pltpu.VMEM((IB, CH), jnp.int32),
        pltpu.VMEM((IB, CH), jnp.int32),
        pltpu.VMEM((IB, CH), jnp.int32),
        pltpu.VMEM((IB, CH), jnp.int32),
        pltpu.VMEM((IB, CH), jnp.int32),
        pltpu.VMEM((KCH1, CH), jnp.int32),
        pltpu.VMEM((KCH1, CH), jnp.int32),
        pltpu.VMEM((CH, D), jnp.float32),
        pltpu.VMEM((CH, D), jnp.float32),
        pltpu.VMEM_SHARED((NPAD, D), jnp.float32),
        pltpu.SemaphoreType.DMA,
        pltpu.SemaphoreType.DMA,
        pltpu.SemaphoreType.DMA,
        pltpu.SemaphoreType.DMA,
        pltpu.SemaphoreType.DMA,
        pltpu.SemaphoreType.DMA,
        pltpu.SemaphoreType.DMA,
        pltpu.SemaphoreType.DMA,
    ],
)
def _scatter_kernel(g_hbm, src_hbm, dst_hbm, out_hbm,
                    sA, sB, sC, dA, dB, dC, s1, d1, rows0, rows1, acc,
                    gsem0, gsem1, ssemA, ssemB, ssemC, dsemA, dsemB, dsemC):
    c = lax.axis_index("c")
    s = lax.axis_index("s")
    rows = [rows0, rows1]
    gsems = [gsem0, gsem1]
    sbufs = [sA, sB, sC]
    ssems = [ssemA, ssemB, ssemC]
    dbufs = [dA, dB, dC]
    dsems = [dsemA, dsemB, dsemC]

    @pl.loop(0, CH)
    def _(i):
        @pl.loop(0, D, step=16)
        def _(jj):
            rows0[i, pl.ds(jj, 16)] = jnp.zeros((16,), jnp.float32)

    @pl.loop(0, NPAD // NS // CH)
    def _(k):
        pltpu.sync_copy(rows0, acc.at[pl.ds(s * (NPAD // NS) + k * CH, CH)])

    pltpu.sync_copy(rows0, acc.at[pl.ds(s * (NPAD // NS) + NPAD // NS - CH, CH)])

    plsc.subcore_barrier()

    def _guard(cond, fn):
        if isinstance(cond, bool):
            if cond:
                fn()
        else:
            pl.when(cond)(fn)

    def _edges_path(cb, kch):
        nblk = kch // IB

        def sload(blk, bi):
            pltpu.async_copy(src_hbm.at[pl.ds(cb + blk * IB, IB)],
                             sbufs[bi], ssems[bi])

        def swait(blk, bi):
            pltpu.make_async_copy(src_hbm.at[pl.ds(cb + blk * IB, IB)],
                                  sbufs[bi], ssems[bi]).wait()

        def dload(blk, bi):
            pltpu.async_copy(dst_hbm.at[pl.ds(cb + blk * IB, IB)],
                             dbufs[bi], dsems[bi])

        def dwait(blk, bi):
            pltpu.make_async_copy(dst_hbm.at[pl.ds(cb + blk * IB, IB)],
                                  dbufs[bi], dsems[bi]).wait()

        def do_block(blk, bi):
            cur_s, next_s = sbufs[bi], sbufs[(bi + 1) % 3]
            cur_d = dbufs[bi]

            _guard(blk + 1 < nblk,
                   lambda: (swait(blk + 1, (bi + 1) % 3),
                            dwait(blk + 1, (bi + 1) % 3)))
            _guard(blk + 2 < nblk,
                   lambda: (sload(blk + 2, (bi + 2) % 3),
                            dload(blk + 2, (bi + 2) % 3)))

            @pl.loop(0, (IB - 2) // 2)
            def _(jj, cur_s=cur_s, cur_d=cur_d):
                for b in range(2):
                    i = jj * 2 + b
                    pltpu.make_async_copy(g_hbm.at[cur_s.at[i]], rows[b],
                                          gsems[b]).wait()
                    pltpu.sync_copy(rows[b], acc.at[cur_d.at[i]], add=True)
                    pltpu.async_copy(g_hbm.at[cur_s.at[i + 2]], rows[b],
                                     gsems[b])

            for i in (IB - 2, IB - 1):
                b = i % 2
                pltpu.make_async_copy(g_hbm.at[cur_s.at[i]], rows[b],
                                      gsems[b]).wait()
                pltpu.sync_copy(rows[b], acc.at[cur_d.at[i]], add=True)
                _guard(blk * IB + i + 2 < kch,
                       lambda i=i, b=b, next_s=next_s: pltpu.async_copy(
                           g_hbm.at[next_s.at[i + 2 - IB]], rows[b],
                           gsems[b]))

        sload(0, 0)
        sload(1, 1)
        dload(0, 0)
        dload(1, 1)
        swait(0, 0)
        pltpu.async_copy(g_hbm.at[sbufs[0].at[0]], rows0, gsem0)
        pltpu.async_copy(g_hbm.at[sbufs[0].at[1]], rows1, gsem1)
        dwait(0, 0)

        @pl.loop(0, nblk // 3)
        def _(gg):
            for bi in range(3):
                do_block(gg * 3 + bi, bi)

        for l in range(nblk % 3):
            blk = nblk - (nblk % 3) + l
            do_block(blk, blk % 3)

    @pl.when(c == 0)
    def _():
        _edges_path(C1ROWS + s * KCH0, KCH0)

    @pl.when(c == 1)
    def _():
        pltpu.sync_copy(src_hbm.at[pl.ds(s * KCH1, KCH1)], s1)
        pltpu.sync_copy(dst_hbm.at[pl.ds(s * KCH1, KCH1)], d1)
        pltpu.async_copy(g_hbm.at[s1.at[0]], rows0, gsem0)
        pltpu.async_copy(g_hbm.at[s1.at[1]], rows1, gsem1)
        for j in range(KCH1):
            b = j % 2
            pltpu.make_async_copy(g_hbm.at[s1.at[j]], rows[b],
                                  gsems[b]).wait()
            pltpu.sync_copy(rows[b], acc.at[d1.at[j]], add=True)
            if j + 2 < KCH1:
                pltpu.async_copy(g_hbm.at[s1.at[j + 2]], rows[b], gsems[b])

    plsc.subcore_barrier()

    @pl.when(s < NS - 1)
    def _():
        pltpu.sync_copy(acc.at[pl.ds(s * RPT, RPT)],
                        out_hbm.at[c, pl.ds(s * RPT, RPT)])

    @pl.when(s == NS - 1)
    def _():
        pltpu.sync_copy(acc.at[pl.ds((NS - 1) * RPT, RPT_LAST)],
                        out_hbm.at[c, pl.ds((NS - 1) * RPT, RPT_LAST)])


@functools.partial(
    pl.kernel,
    out_type=jax.ShapeDtypeStruct((NC, N, D), jnp.float32),
    mesh=_mesh,
    scratch_types=[
        pltpu.VMEM((IB, CH), jnp.int32),
        pltpu.VMEM((IB, CH), jnp.int32),
        pltpu.VMEM((IB, CH), jnp.int32),
        pltpu.VMEM((IB, CH), jnp.int32),
        pltpu.VMEM((IB, CH), jnp.int32),
        pltpu.VMEM((IB, CH), jnp.int32),
        pltpu.VMEM((KCH1, CH), jnp.int32),
        pltpu.VMEM((KCH1, CH), jnp.int32),
        pltpu.VMEM((CH, D), jnp.float32),
        pltpu.VMEM((CH, D), jnp.float32),
        pltpu.VMEM_SHARED((NPAD, D), jnp.float32),
        pltpu.SemaphoreType.DMA,
        pltpu.SemaphoreType.DMA,
        pltpu.SemaphoreType.DMA,
        pltpu.SemaphoreType.DMA,
        pltpu.SemaphoreType.DMA,
        pltpu.SemaphoreType.DMA,
        pltpu.SemaphoreType.DMA,
        pltpu.SemaphoreType.DMA,
    ],
)
def _scatter_kernel(g_hbm, src_hbm, dst_hbm, out_hbm,
                    sA, sB, sC, dA, dB, dC, s1, d1, rows0, rows1, acc,
                    gsem0, gsem1, ssemA, ssemB, ssemC, dsemA, dsemB, dsemC):
    c = lax.axis_index("c")
    s = lax.axis_index("s")
    rows = [rows0, rows1]
    gsems = [gsem0, gsem1]
    sbufs = [sA, sB, sC]
    ssems = [ssemA, ssemB, ssemC]
    dbufs = [dA, dB, dC]
    dsems = [dsemA, dsemB, dsemC]

    @pl.loop(0, CH)
    def _(i):
        @pl.loop(0, D, step=16)
        def _(jj):
            rows0[i, pl.ds(jj, 16)] = jnp.zeros((16,), jnp.float32)

    @pl.loop(0, NPAD // NS // CH)
    def _(k):
        pltpu.sync_copy(rows0, acc.at[pl.ds(s * (NPAD // NS) + k * CH, CH)])

    pltpu.sync_copy(rows0, acc.at[pl.ds(s * (NPAD // NS) + NPAD // NS - CH, CH)])

    plsc.subcore_barrier()

    @pl.when(c == 0)
    def _():
        cb = C1ROWS + s * KCH0

        def sload(blk, bi):
            pltpu.async_copy(src_hbm.at[pl.ds(cb + blk * IB, IB)],
                             sbufs[bi], ssems[bi])

        def swait(blk, bi):
            pltpu.make_async_copy(src_hbm.at[pl.ds(cb + blk * IB, IB)],
                                  sbufs[bi], ssems[bi]).wait()

        def dload(blk, bi):
            pltpu.async_copy(dst_hbm.at[pl.ds(cb + blk * IB, IB)],
                             dbufs[bi], dsems[bi])

        def dwait(blk, bi):
            pltpu.make_async_copy(dst_hbm.at[pl.ds(cb + blk * IB, IB)],
                                  dbufs[bi], dsems[bi]).wait()

        sload(0, 0)
        sload(1, 1)
        dload(0, 0)
        dload(1, 1)
        swait(0, 0)
        pltpu.async_copy(g_hbm.at[sA.at[0]], rows0, gsem0)
        pltpu.async_copy(g_hbm.at[sA.at[1]], rows1, gsem1)
        dwait(0, 0)

        @pl.loop(0, NBLK0 // 3)
        def _(gg):
            for bi in range(3):
                blk = gg * 3 + bi
                cur_s, next_s = sbufs[bi], sbufs[(bi + 1) % 3]
                cur_d = dbufs[bi]

                @pl.when(blk + 1 < NBLK0)
                def _(blk=blk, bi=bi):
                    swait(blk + 1, (bi + 1) % 3)
                    dwait(blk + 1, (bi + 1) % 3)

                @pl.when(blk + 2 < NBLK0)
                def _(blk=blk, bi=bi):
                    sload(blk + 2, (bi + 2) % 3)
                    dload(blk + 2, (bi + 2) % 3)

                @pl.loop(0, (IB - 2) // 2)
                def _(jj, cur_s=cur_s, cur_d=cur_d):
                    for b in range(2):
                        i = jj * 2 + b
                        pltpu.make_async_copy(g_hbm.at[cur_s.at[i]], rows[b],
                                              gsems[b]).wait()
                        pltpu.sync_copy(rows[b], acc.at[cur_d.at[i]],
                                        add=True)
                        pltpu.async_copy(g_hbm.at[cur_s.at[i + 2]], rows[b],
                                         gsems[b])

                for i in (IB - 2, IB - 1):
                    b = i % 2
                    pltpu.make_async_copy(g_hbm.at[cur_s.at[i]], rows[b],
                                          gsems[b]).wait()
                    pltpu.sync_copy(rows[b], acc.at[cur_d.at[i]], add=True)

                    @pl.when(blk * IB + i + 2 < KCH0)
                    def _(i=i, b=b, next_s=next_s):
                        pltpu.async_copy(g_hbm.at[next_s.at[i + 2 - IB]],
                                         rows[b], gsems[b])

    @pl.when(c == 1)
    def _():
        pltpu.sync_copy(src_hbm.at[pl.ds(s * KCH1, KCH1)], s1)
        pltpu.sync_copy(dst_hbm.at[pl.ds(s * KCH1, KCH1)], d1)
        pltpu.async_copy(g_hbm.at[s1.at[0]], rows0, gsem0)
        pltpu.async_copy(g_hbm.at[s1.at[1]], rows1, gsem1)
        for j in range(KCH1):
            b = j % 2
            pltpu.make_async_copy(g_hbm.at[s1.at[j]], rows[b],
                                  gsems[b]).wait()
            pltpu.sync_copy(rows[b], acc.at[d1.at[j]], add=True)
            if j + 2 < KCH1:
                pltpu.async_copy(g_hbm.at[s1.at[j + 2]], rows[b], gsems[b])

    plsc.subcore_barrier()

    @pl.when(s < NS - 1)
    def _():
        pltpu.sync_copy(acc.at[pl.ds(s * RPT, RPT)],
                        out_hbm.at[c, pl.ds(s * RPT, RPT)])

    @pl.when(s == NS - 1)
    def _():
        pltpu.sync_copy(acc.at[pl.ds((NS - 1) * RPT, RPT_LAST)],
                        out_hbm.at[c, pl.ds((NS - 1) * RPT, RPT_LAST)])


def _dis(dp0, dp1):
    deg = dp0[:, :1] + dp1[:, :1] + 1.0
    return lax.rsqrt(jnp.maximum(deg, 1.0))


def _mm_scale_body(x_ref, w_ref, dp0_ref, dp1_ref, o_ref):
    o_ref[...] = _dis(dp0_ref[0], dp1_ref[0]) * jnp.dot(
        x_ref[...], w_ref[...], preferred_element_type=jnp.float32)


def _mm_scale(x, w, degp):
    return pl.pallas_call(
        _mm_scale_body,
        grid=(N // RB,),
        in_specs=[pl.BlockSpec((RB, D), lambda i: (i, 0)),
                  pl.BlockSpec((D, D), lambda i: (0, 0)),
                  pl.BlockSpec((1, RB, D), lambda i: (0, i, 0)),
                  pl.BlockSpec((1, RB, D), lambda i: (1, i, 0))],
        out_specs=pl.BlockSpec((RB, D), lambda i: (i, 0)),
        out_shape=jax.ShapeDtypeStruct((N, D), jnp.float32),
    )(x, w, degp, degp)


def _layer_body(p0_ref, p1_ref, g_ref, dp0_ref, dp1_ref, b_ref, w_ref, o_ref):
    dis = _dis(dp0_ref[0], dp1_ref[0])
    h = jnp.maximum(
        dis * (p0_ref[0] + p1_ref[0] + g_ref[...]) + b_ref[...], 0.0)
    o_ref[...] = dis * jnp.dot(h, w_ref[...],
                               preferred_element_type=jnp.float32)


def _layer(p, g, degp, b, w):
    return pl.pallas_call(
        _layer_body,
        grid=(N // RB,),
        in_specs=[pl.BlockSpec((1, RB, D), lambda i: (0, i, 0)),
                  pl.BlockSpec((1, RB, D), lambda i: (1, i, 0)),
                  pl.BlockSpec((RB, D), lambda i: (i, 0)),
                  pl.BlockSpec((1, RB, D), lambda i: (0, i, 0)),
                  pl.BlockSpec((1, RB, D), lambda i: (1, i, 0)),
                  pl.BlockSpec((1, D), lambda i: (0, 0)),
                  pl.BlockSpec((D, D), lambda i: (0, 0))],
        out_specs=pl.BlockSpec((RB, D), lambda i: (i, 0)),
        out_shape=jax.ShapeDtypeStruct((N, D), jnp.float32),
    )(p, p, g, degp, degp, b, w)


def _final_body(p0_ref, p1_ref, g_ref, dp0_ref, dp1_ref, b_ref, bat_ref,
                l1w_ref, l1b_ref, l2w_ref, l2b_ref, o_ref, sums, cnts):
    i = pl.program_id(0)

    @pl.when(i == 0)
    def _():
        sums[...] = jnp.zeros_like(sums)
        cnts[...] = jnp.zeros_like(cnts)

    dis = _dis(dp0_ref[0], dp1_ref[0])
    h = jnp.maximum(
        dis * (p0_ref[0] + p1_ref[0] + g_ref[...]) + b_ref[...], 0.0)
    gid = lax.broadcasted_iota(jnp.int32, (RB, G), 1)
    mask = (bat_ref[...] == gid).astype(jnp.float32)
    dn = (((0,), (0,)), ((), ()))
    sums[...] += lax.dot_general(mask, h, dn,
                                 preferred_element_type=jnp.float32)
    cnts[...] += lax.dot_general(mask, jnp.ones((RB, D), jnp.float32), dn,
                                 preferred_element_type=jnp.float32)

    @pl.when(i == (N // RB) - 1)
    def _():
        pooled = sums[...] / jnp.maximum(cnts[...], 1.0)
        z = jnp.maximum(
            jnp.dot(pooled, l1w_ref[...],
                    preferred_element_type=jnp.float32) + l1b_ref[...], 0.0)
        o_ref[...] = jnp.dot(z, l2w_ref[...],
                             preferred_element_type=jnp.float32) + l2b_ref[...]


def _final(p, g, degp, b, bat, l1w, l1b, l2w, l2b):
    return pl.pallas_call(
        _final_body,
        grid=(N // RB,),
        in_specs=[pl.BlockSpec((1, RB, D), lambda i: (0, i, 0)),
                  pl.BlockSpec((1, RB, D), lambda i: (1, i, 0)),
                  pl.BlockSpec((RB, D), lambda i: (i, 0)),
                  pl.BlockSpec((1, RB, D), lambda i: (0, i, 0)),
                  pl.BlockSpec((1, RB, D), lambda i: (1, i, 0)),
                  pl.BlockSpec((1, D), lambda i: (0, 0)),
                  pl.BlockSpec((RB, 1), lambda i: (i, 0)),
                  pl.BlockSpec((D, D), lambda i: (0, 0)),
                  pl.BlockSpec((1, D), lambda i: (0, 0)),
                  pl.BlockSpec((D, C), lambda i: (0, 0)),
                  pl.BlockSpec((1, C), lambda i: (0, 0))],
        out_specs=pl.BlockSpec((G, C), lambda i: (0, 0)),
        out_shape=jax.ShapeDtypeStruct((G, C), jnp.float32),
        scratch_shapes=[pltpu.VMEM((G, D), jnp.float32),
                        pltpu.VMEM((G, D), jnp.float32)],
    )(p, p, g, degp, degp, b, bat, l1w, l1b, l2w, l2b)


def kernel(x, edge_index, batch, W1, b1, W2, b2, W3, b3,
           lin1_W, lin1_b, lin2_W, lin2_b):
    src = edge_index[0].astype(jnp.int32)
    dst = edge_index[1].astype(jnp.int32)
    pad = EPAD - E
    pidx = jnp.arange(pad, dtype=jnp.int32) % (NPAD - N)
    srcp = jnp.concatenate([src, pidx % CH]).reshape(TOTCH, CH)
    dstp = jnp.concatenate([dst, N + pidx]).reshape(TOTCH, CH)

    degp = _deg_kernel(dstp.reshape(NT, KCH, CH))

    g0 = _mm_scale(x, W1, degp)
    p = _scatter_kernel(g0, srcp, dstp)
    g1 = _layer(p, g0, degp, b1.reshape(1, D), W2)
    p = _scatter_kernel(g1, srcp, dstp)
    g2 = _layer(p, g1, degp, b2.reshape(1, D), W3)
    p = _scatter_kernel(g2, srcp, dstp)
    return _final(p, g2, degp, b3.reshape(1, D),
                  batch.reshape(N, 1).astype(jnp.int32),
                  lin1_W, lin1_b.reshape(1, D), lin2_W, lin2_b.reshape(1, C))


# speedup vs baseline: 1.0933x; 1.0933x over previous
import functools

import jax
import jax.numpy as jnp
from jax import lax
from jax.experimental import pallas as pl
from jax.experimental.pallas import tpu as pltpu
from jax.experimental.pallas import tpu_sc as plsc

N = 10000
D = 128
E = 320000
G = 64
C = 16

NC = 2
NS = 16
NT = NC * NS
CH = 128
KCH = 80
IB = 8
KCH0 = 104
KCH1 = 40
KX1 = 16
NBLK0 = KCH0 // IB
C1ROWS = NS * (KCH1 + KX1)
TOTCH = NS * (KCH0 + KCH1 + KX1)
EPAD = TOTCH * CH
NPAD = 10112
RPT = 624
RPT_LAST = N - RPT * (NS - 1)
ZCH = NPAD // NS // CH

RB = 1000

_mesh = plsc.VectorSubcoreMesh(core_axis_name="c", subcore_axis_name="s")


@functools.partial(
    pl.kernel,
    out_type=jax.ShapeDtypeStruct((NC, N, D), jnp.float32),
    mesh=_mesh,
    scratch_types=[
        pltpu.VMEM((KCH, CH), jnp.int32),
        pltpu.VMEM((CH, D), jnp.float32),
        pltpu.VMEM((CH, D), jnp.float32),
        pltpu.VMEM_SHARED((NPAD, D), jnp.float32),
    ],
)
def _deg_kernel(dst_hbm, out_hbm, didx, ones, zeros, acc):
    c = lax.axis_index("c")
    s = lax.axis_index("s")
    t = c * NS + s

    @pl.loop(0, CH)
    def _(i):
        @pl.loop(0, D, step=16)
        def _(jj):
            ones[i, pl.ds(jj, 16)] = jnp.ones((16,), jnp.float32)
            zeros[i, pl.ds(jj, 16)] = jnp.zeros((16,), jnp.float32)

    @pl.loop(0, NPAD // NS // CH)
    def _(k):
        pltpu.sync_copy(zeros, acc.at[pl.ds(s * (NPAD // NS) + k * CH, CH)])

    pltpu.sync_copy(zeros, acc.at[pl.ds(s * (NPAD // NS) + NPAD // NS - CH, CH)])

    plsc.subcore_barrier()

    pltpu.sync_copy(dst_hbm.at[t], didx)

    @pl.loop(0, KCH)
    def _(j):
        pltpu.sync_copy(ones, acc.at[didx.at[j]], add=True)

    plsc.subcore_barrier()

    @pl.when(s < NS - 1)
    def _():
        pltpu.sync_copy(acc.at[pl.ds(s * RPT, RPT)],
                        out_hbm.at[c, pl.ds(s * RPT, RPT)])

    @pl.when(s == NS - 1)
    def _():
        pltpu.sync_copy(acc.at[pl.ds((NS - 1) * RPT, RPT_LAST)],
                        out_hbm.at[c, pl.ds((NS - 1) * RPT, RPT_LAST)])


@functools.partial(
    pl.kernel,
    out_type=jax.ShapeDtypeStruct((NC, N, D), jnp.float32),
    mesh=_mesh,
    scratch_types=[
        pltpu.VMEM((IB, CH), jnp.int32),
        pltpu.VMEM((IB, CH), jnp.int32),
        pltpu.VMEM((IB, CH), jnp.int32),
        pltpu.VMEM((IB, CH), jnp.int32),
        pltpu.VMEM((IB, CH), jnp.int32),
        pltpu.VMEM((IB, CH), jnp.int32),
        pltpu.VMEM((KCH1, CH), jnp.int32),
        pltpu.VMEM((KCH1, CH), jnp.int32),
        pltpu.VMEM((CH, D), jnp.float32),
        pltpu.VMEM((CH, D), jnp.float32),
        pltpu.VMEM_SHARED((NPAD, D), jnp.float32),
        pltpu.SemaphoreType.DMA,
        pltpu.SemaphoreType.DMA,
        pltpu.SemaphoreType.DMA,
        pltpu.SemaphoreType.DMA,
        pltpu.SemaphoreType.DMA,
        pltpu.SemaphoreType.DMA,
        pltpu.SemaphoreType.DMA,
        pltpu.SemaphoreType.DMA,
    ],
)
def _scatter_kernel(g_hbm, src_hbm, dst_hbm, out_hbm,
                    sA, sB, sC, dA, dB, dC, s1, d1, rows0, rows1, acc,
                    gsem0, gsem1, ssemA, ssemB, ssemC, dsemA, dsemB, dsemC):
    c = lax.axis_index("c")
    s = lax.axis_index("s")
    rows = [rows0, rows1]
    gsems = [gsem0, gsem1]
    sbufs = [sA, sB, sC]
    ssems = [ssemA, ssemB, ssemC]
    dbufs = [dA, dB, dC]
    dsems = [dsemA, dsemB, dsemC]

    @pl.loop(0, CH)
    def _(i):
        @pl.loop(0, D, step=16)
        def _(jj):
            rows0[i, pl.ds(jj, 16)] = jnp.zeros((16,), jnp.float32)

    @pl.loop(0, NPAD // NS // CH)
    def _(k):
        pltpu.sync_copy(rows0, acc.at[pl.ds(s * (NPAD // NS) + k * CH, CH)])

    pltpu.sync_copy(rows0, acc.at[pl.ds(s * (NPAD // NS) + NPAD // NS - CH, CH)])

    plsc.subcore_barrier()

    def _guard(cond, fn):
        if isinstance(cond, bool):
            if cond:
                fn()
        else:
            pl.when(cond)(fn)

    def _edges_path(cb, kch):
        nblk = kch // IB

        def sload(blk, bi):
            pltpu.async_copy(src_hbm.at[pl.ds(cb + blk * IB, IB)],
                             sbufs[bi], ssems[bi])

        def swait(blk, bi):
            pltpu.make_async_copy(src_hbm.at[pl.ds(cb + blk * IB, IB)],
                                  sbufs[bi], ssems[bi]).wait()

        def dload(blk, bi):
            pltpu.async_copy(dst_hbm.at[pl.ds(cb + blk * IB, IB)],
                             dbufs[bi], dsems[bi])

        def dwait(blk, bi):
            pltpu.make_async_copy(dst_hbm.at[pl.ds(cb + blk * IB, IB)],
                                  dbufs[bi], dsems[bi]).wait()

        def do_block(blk, bi):
            cur_s, next_s = sbufs[bi], sbufs[(bi + 1) % 3]
            cur_d = dbufs[bi]

            def _wait_next(blk=blk, bi=bi):
                swait(blk + 1, (bi + 1) % 3)
                dwait(blk + 1, (bi + 1) % 3)

            def _load_next2(blk=blk, bi=bi):
                sload(blk + 2, (bi + 2) % 3)
                dload(blk + 2, (bi + 2) % 3)

            _guard(blk + 1 < nblk, _wait_next)
            _guard(blk + 2 < nblk, _load_next2)

            @pl.loop(0, (IB - 2) // 2)
            def _(jj, cur_s=cur_s, cur_d=cur_d):
                for b in range(2):
                    i = jj * 2 + b
                    pltpu.make_async_copy(g_hbm.at[cur_s.at[i]], rows[b],
                                          gsems[b]).wait()
                    pltpu.sync_copy(rows[b], acc.at[cur_d.at[i]], add=True)
                    pltpu.async_copy(g_hbm.at[cur_s.at[i + 2]], rows[b],
                                     gsems[b])

            for i in (IB - 2, IB - 1):
                b = i % 2
                pltpu.make_async_copy(g_hbm.at[cur_s.at[i]], rows[b],
                                      gsems[b]).wait()
                pltpu.sync_copy(rows[b], acc.at[cur_d.at[i]], add=True)
                def _refire(i=i, b=b, next_s=next_s):
                    pltpu.async_copy(g_hbm.at[next_s.at[i + 2 - IB]],
                                     rows[b], gsems[b])

                _guard(blk * IB + i + 2 < kch, _refire)

        sload(0, 0)
        sload(1, 1)
        dload(0, 0)
        dload(1, 1)
        swait(0, 0)
        pltpu.async_copy(g_hbm.at[sbufs[0].at[0]], rows0, gsem0)
        pltpu.async_copy(g_hbm.at[sbufs[0].at[1]], rows1, gsem1)
        dwait(0, 0)

        @pl.loop(0, nblk // 3)
        def _(gg):
            for bi in range(3):
                do_block(gg * 3 + bi, bi)

        for l in range(nblk % 3):
            blk = nblk - (nblk % 3) + l
            do_block(blk, blk % 3)

    @pl.when(c == 0)
    def _():
        _edges_path(C1ROWS + s * KCH0, KCH0)

    @pl.when(c == 1)
    def _():
        base1 = s * (KCH1 + KX1)
        pltpu.sync_copy(src_hbm.at[pl.ds(base1, KCH1)], s1)
        pltpu.sync_copy(dst_hbm.at[pl.ds(base1, KCH1)], d1)
        pltpu.async_copy(g_hbm.at[s1.at[0]], rows0, gsem0)
        pltpu.async_copy(g_hbm.at[s1.at[1]], rows1, gsem1)
        for j in range(KCH1):
            b = j % 2
            pltpu.make_async_copy(g_hbm.at[s1.at[j]], rows[b],
                                  gsems[b]).wait()
            pltpu.sync_copy(rows[b], acc.at[d1.at[j]], add=True)
            if j + 2 < KCH1:
                pltpu.async_copy(g_hbm.at[s1.at[j + 2]], rows[b], gsems[b])
        _edges_path(base1 + KCH1, KX1)

    plsc.subcore_barrier()

    @pl.when(s < NS - 1)
    def _():
        pltpu.sync_copy(acc.at[pl.ds(s * RPT, RPT)],
                        out_hbm.at[c, pl.ds(s * RPT, RPT)])

    @pl.when(s == NS - 1)
    def _():
        pltpu.sync_copy(acc.at[pl.ds((NS - 1) * RPT, RPT_LAST)],
                        out_hbm.at[c, pl.ds((NS - 1) * RPT, RPT_LAST)])


def _dis(dp0, dp1):
    deg = dp0[:, :1] + dp1[:, :1] + 1.0
    return lax.rsqrt(jnp.maximum(deg, 1.0))


def _mm_scale_body(x_ref, w_ref, dp0_ref, dp1_ref, o_ref):
    o_ref[...] = _dis(dp0_ref[0], dp1_ref[0]) * jnp.dot(
        x_ref[...], w_ref[...], preferred_element_type=jnp.float32)


def _mm_scale(x, w, degp):
    return pl.pallas_call(
        _mm_scale_body,
        grid=(N // RB,),
        in_specs=[pl.BlockSpec((RB, D), lambda i: (i, 0)),
                  pl.BlockSpec((D, D), lambda i: (0, 0)),
                  pl.BlockSpec((1, RB, D), lambda i: (0, i, 0)),
                  pl.BlockSpec((1, RB, D), lambda i: (1, i, 0))],
        out_specs=pl.BlockSpec((RB, D), lambda i: (i, 0)),
        out_shape=jax.ShapeDtypeStruct((N, D), jnp.float32),
    )(x, w, degp, degp)


def _layer_body(p0_ref, p1_ref, g_ref, dp0_ref, dp1_ref, b_ref, w_ref, o_ref):
    dis = _dis(dp0_ref[0], dp1_ref[0])
    h = jnp.maximum(
        dis * (p0_ref[0] + p1_ref[0] + g_ref[...]) + b_ref[...], 0.0)
    o_ref[...] = dis * jnp.dot(h, w_ref[...],
                               preferred_element_type=jnp.float32)


def _layer(p, g, degp, b, w):
    return pl.pallas_call(
        _layer_body,
        grid=(N // RB,),
        in_specs=[pl.BlockSpec((1, RB, D), lambda i: (0, i, 0)),
                  pl.BlockSpec((1, RB, D), lambda i: (1, i, 0)),
                  pl.BlockSpec((RB, D), lambda i: (i, 0)),
                  pl.BlockSpec((1, RB, D), lambda i: (0, i, 0)),
                  pl.BlockSpec((1, RB, D), lambda i: (1, i, 0)),
                  pl.BlockSpec((1, D), lambda i: (0, 0)),
                  pl.BlockSpec((D, D), lambda i: (0, 0))],
        out_specs=pl.BlockSpec((RB, D), lambda i: (i, 0)),
        out_shape=jax.ShapeDtypeStruct((N, D), jnp.float32),
    )(p, p, g, degp, degp, b, w)


def _final_body(p0_ref, p1_ref, g_ref, dp0_ref, dp1_ref, b_ref, bat_ref,
                l1w_ref, l1b_ref, l2w_ref, l2b_ref, o_ref, sums, cnts):
    i = pl.program_id(0)

    @pl.when(i == 0)
    def _():
        sums[...] = jnp.zeros_like(sums)
        cnts[...] = jnp.zeros_like(cnts)

    dis = _dis(dp0_ref[0], dp1_ref[0])
    h = jnp.maximum(
        dis * (p0_ref[0] + p1_ref[0] + g_ref[...]) + b_ref[...], 0.0)
    gid = lax.broadcasted_iota(jnp.int32, (RB, G), 1)
    mask = (bat_ref[...] == gid).astype(jnp.float32)
    dn = (((0,), (0,)), ((), ()))
    sums[...] += lax.dot_general(mask, h, dn,
                                 preferred_element_type=jnp.float32)
    cnts[...] += lax.dot_general(mask, jnp.ones((RB, D), jnp.float32), dn,
                                 preferred_element_type=jnp.float32)

    @pl.when(i == (N // RB) - 1)
    def _():
        pooled = sums[...] / jnp.maximum(cnts[...], 1.0)
        z = jnp.maximum(
            jnp.dot(pooled, l1w_ref[...],
                    preferred_element_type=jnp.float32) + l1b_ref[...], 0.0)
        o_ref[...] = jnp.dot(z, l2w_ref[...],
                             preferred_element_type=jnp.float32) + l2b_ref[...]


def _final(p, g, degp, b, bat, l1w, l1b, l2w, l2b):
    return pl.pallas_call(
        _final_body,
        grid=(N // RB,),
        in_specs=[pl.BlockSpec((1, RB, D), lambda i: (0, i, 0)),
                  pl.BlockSpec((1, RB, D), lambda i: (1, i, 0)),
                  pl.BlockSpec((RB, D), lambda i: (i, 0)),
                  pl.BlockSpec((1, RB, D), lambda i: (0, i, 0)),
                  pl.BlockSpec((1, RB, D), lambda i: (1, i, 0)),
                  pl.BlockSpec((1, D), lambda i: (0, 0)),
                  pl.BlockSpec((RB, 1), lambda i: (i, 0)),
                  pl.BlockSpec((D, D), lambda i: (0, 0)),
                  pl.BlockSpec((1, D), lambda i: (0, 0)),
                  pl.BlockSpec((D, C), lambda i: (0, 0)),
                  pl.BlockSpec((1, C), lambda i: (0, 0))],
        out_specs=pl.BlockSpec((G, C), lambda i: (0, 0)),
        out_shape=jax.ShapeDtypeStruct((G, C), jnp.float32),
        scratch_shapes=[pltpu.VMEM((G, D), jnp.float32),
                        pltpu.VMEM((G, D), jnp.float32)],
    )(p, p, g, degp, degp, b, bat, l1w, l1b, l2w, l2b)


def kernel(x, edge_index, batch, W1, b1, W2, b2, W3, b3,
           lin1_W, lin1_b, lin2_W, lin2_b):
    src = edge_index[0].astype(jnp.int32)
    dst = edge_index[1].astype(jnp.int32)
    pad = EPAD - E
    pidx = jnp.arange(pad, dtype=jnp.int32) % (NPAD - N)
    srcp = jnp.concatenate([src, pidx % CH]).reshape(TOTCH, CH)
    dstp = jnp.concatenate([dst, N + pidx]).reshape(TOTCH, CH)

    degp = _deg_kernel(dstp.reshape(NT, KCH, CH))

    g0 = _mm_scale(x, W1, degp)
    p = _scatter_kernel(g0, srcp, dstp)
    g1 = _layer(p, g0, degp, b1.reshape(1, D), W2)
    p = _scatter_kernel(g1, srcp, dstp)
    g2 = _layer(p, g1, degp, b2.reshape(1, D), W3)
    p = _scatter_kernel(g2, srcp, dstp)
    return _final(p, g2, degp, b3.reshape(1, D),
                  batch.reshape(N, 1).astype(jnp.int32),
                  lin1_W, lin1_b.reshape(1, D), lin2_W, lin2_b.reshape(1, C))


# speedup vs baseline: 1.1534x; 1.0550x over previous
import functools

import jax
import jax.numpy as jnp
from jax import lax
from jax.experimental import pallas as pl
from jax.experimental.pallas import tpu as pltpu
from jax.experimental.pallas import tpu_sc as plsc

N = 10000
D = 128
E = 320000
G = 64
C = 16

NC = 2
NS = 16
NT = NC * NS
CH = 128
KCH = 80
IB = 8
KCH0 = 96
KCH1 = 40
KX1 = 24
NBLK0 = KCH0 // IB
C1ROWS = NS * (KCH1 + KX1)
TOTCH = NS * (KCH0 + KCH1 + KX1)
EPAD = TOTCH * CH
NPAD = 10112
RPT = 624
RPT_LAST = N - RPT * (NS - 1)
ZCH = NPAD // NS // CH

RB = 1000

_mesh = plsc.VectorSubcoreMesh(core_axis_name="c", subcore_axis_name="s")


@functools.partial(
    pl.kernel,
    out_type=jax.ShapeDtypeStruct((NC, N, D), jnp.float32),
    mesh=_mesh,
    scratch_types=[
        pltpu.VMEM((KCH, CH), jnp.int32),
        pltpu.VMEM((CH, D), jnp.float32),
        pltpu.VMEM((CH, D), jnp.float32),
        pltpu.VMEM_SHARED((NPAD, D), jnp.float32),
    ],
)
def _deg_kernel(dst_hbm, out_hbm, didx, ones, zeros, acc):
    c = lax.axis_index("c")
    s = lax.axis_index("s")
    t = c * NS + s

    @pl.loop(0, CH)
    def _(i):
        @pl.loop(0, D, step=16)
        def _(jj):
            ones[i, pl.ds(jj, 16)] = jnp.ones((16,), jnp.float32)
            zeros[i, pl.ds(jj, 16)] = jnp.zeros((16,), jnp.float32)

    @pl.loop(0, NPAD // NS // CH)
    def _(k):
        pltpu.sync_copy(zeros, acc.at[pl.ds(s * (NPAD // NS) + k * CH, CH)])

    pltpu.sync_copy(zeros, acc.at[pl.ds(s * (NPAD // NS) + NPAD // NS - CH, CH)])

    plsc.subcore_barrier()

    pltpu.sync_copy(dst_hbm.at[t], didx)

    @pl.loop(0, KCH)
    def _(j):
        pltpu.sync_copy(ones, acc.at[didx.at[j]], add=True)

    plsc.subcore_barrier()

    @pl.when(s < NS - 1)
    def _():
        pltpu.sync_copy(acc.at[pl.ds(s * RPT, RPT)],
                        out_hbm.at[c, pl.ds(s * RPT, RPT)])

    @pl.when(s == NS - 1)
    def _():
        pltpu.sync_copy(acc.at[pl.ds((NS - 1) * RPT, RPT_LAST)],
                        out_hbm.at[c, pl.ds((NS - 1) * RPT, RPT_LAST)])


@functools.partial(
    pl.kernel,
    out_type=jax.ShapeDtypeStruct((NC, N, D), jnp.float32),
    mesh=_mesh,
    scratch_types=[
        pltpu.VMEM((IB, CH), jnp.int32),
        pltpu.VMEM((IB, CH), jnp.int32),
        pltpu.VMEM((IB, CH), jnp.int32),
        pltpu.VMEM((IB, CH), jnp.int32),
        pltpu.VMEM((IB, CH), jnp.int32),
        pltpu.VMEM((IB, CH), jnp.int32),
        pltpu.VMEM((KCH1, CH), jnp.int32),
        pltpu.VMEM((KCH1, CH), jnp.int32),
        pltpu.VMEM((CH, D), jnp.float32),
        pltpu.VMEM((CH, D), jnp.float32),
        pltpu.VMEM_SHARED((NPAD, D), jnp.float32),
        pltpu.SemaphoreType.DMA,
        pltpu.SemaphoreType.DMA,
        pltpu.SemaphoreType.DMA,
        pltpu.SemaphoreType.DMA,
        pltpu.SemaphoreType.DMA,
        pltpu.SemaphoreType.DMA,
        pltpu.SemaphoreType.DMA,
        pltpu.SemaphoreType.DMA,
    ],
)
def _scatter_kernel(g_hbm, src_hbm, dst_hbm, out_hbm,
                    sA, sB, sC, dA, dB, dC, s1, d1, rows0, rows1, acc,
                    gsem0, gsem1, ssemA, ssemB, ssemC, dsemA, dsemB, dsemC):
    c = lax.axis_index("c")
    s = lax.axis_index("s")
    rows = [rows0, rows1]
    gsems = [gsem0, gsem1]
    sbufs = [sA, sB, sC]
    ssems = [ssemA, ssemB, ssemC]
    dbufs = [dA, dB, dC]
    dsems = [dsemA, dsemB, dsemC]

    @pl.loop(0, CH)
    def _(i):
        @pl.loop(0, D, step=16)
        def _(jj):
            rows0[i, pl.ds(jj, 16)] = jnp.zeros((16,), jnp.float32)

    @pl.loop(0, NPAD // NS // CH)
    def _(k):
        pltpu.sync_copy(rows0, acc.at[pl.ds(s * (NPAD // NS) + k * CH, CH)])

    pltpu.sync_copy(rows0, acc.at[pl.ds(s * (NPAD // NS) + NPAD // NS - CH, CH)])

    plsc.subcore_barrier()

    def _guard(cond, fn):
        if isinstance(cond, bool):
            if cond:
                fn()
        else:
            pl.when(cond)(fn)

    def _edges_path(cb, kch):
        nblk = kch // IB

        def sload(blk, bi):
            pltpu.async_copy(src_hbm.at[pl.ds(cb + blk * IB, IB)],
                             sbufs[bi], ssems[bi])

        def swait(blk, bi):
            pltpu.make_async_copy(src_hbm.at[pl.ds(cb + blk * IB, IB)],
                                  sbufs[bi], ssems[bi]).wait()

        def dload(blk, bi):
            pltpu.async_copy(dst_hbm.at[pl.ds(cb + blk * IB, IB)],
                             dbufs[bi], dsems[bi])

        def dwait(blk, bi):
            pltpu.make_async_copy(dst_hbm.at[pl.ds(cb + blk * IB, IB)],
                                  dbufs[bi], dsems[bi]).wait()

        def do_block(blk, bi):
            cur_s, next_s = sbufs[bi], sbufs[(bi + 1) % 3]
            cur_d = dbufs[bi]

            def _wait_next(blk=blk, bi=bi):
                swait(blk + 1, (bi + 1) % 3)
                dwait(blk + 1, (bi + 1) % 3)

            def _load_next2(blk=blk, bi=bi):
                sload(blk + 2, (bi + 2) % 3)
                dload(blk + 2, (bi + 2) % 3)

            _guard(blk + 1 < nblk, _wait_next)
            _guard(blk + 2 < nblk, _load_next2)

            @pl.loop(0, (IB - 2) // 2)
            def _(jj, cur_s=cur_s, cur_d=cur_d):
                for b in range(2):
                    i = jj * 2 + b
                    pltpu.make_async_copy(g_hbm.at[cur_s.at[i]], rows[b],
                                          gsems[b]).wait()
                    pltpu.sync_copy(rows[b], acc.at[cur_d.at[i]], add=True)
                    pltpu.async_copy(g_hbm.at[cur_s.at[i + 2]], rows[b],
                                     gsems[b])

            for i in (IB - 2, IB - 1):
                b = i % 2
                pltpu.make_async_copy(g_hbm.at[cur_s.at[i]], rows[b],
                                      gsems[b]).wait()
                pltpu.sync_copy(rows[b], acc.at[cur_d.at[i]], add=True)
                def _refire(i=i, b=b, next_s=next_s):
                    pltpu.async_copy(g_hbm.at[next_s.at[i + 2 - IB]],
                                     rows[b], gsems[b])

                _guard(blk * IB + i + 2 < kch, _refire)

        sload(0, 0)
        sload(1, 1)
        dload(0, 0)
        dload(1, 1)
        swait(0, 0)
        pltpu.async_copy(g_hbm.at[sbufs[0].at[0]], rows0, gsem0)
        pltpu.async_copy(g_hbm.at[sbufs[0].at[1]], rows1, gsem1)
        dwait(0, 0)

        @pl.loop(0, nblk // 3)
        def _(gg):
            for bi in range(3):
                do_block(gg * 3 + bi, bi)

        for l in range(nblk % 3):
            blk = nblk - (nblk % 3) + l
            do_block(blk, blk % 3)

    @pl.when(c == 0)
    def _():
        _edges_path(C1ROWS + s * KCH0, KCH0)

    @pl.when(c == 1)
    def _():
        base1 = s * (KCH1 + KX1)
        pltpu.sync_copy(src_hbm.at[pl.ds(base1, KCH1)], s1)
        pltpu.sync_copy(dst_hbm.at[pl.ds(base1, KCH1)], d1)
        pltpu.async_copy(g_hbm.at[s1.at[0]], rows0, gsem0)
        pltpu.async_copy(g_hbm.at[s1.at[1]], rows1, gsem1)
        for j in range(KCH1):
            b = j % 2
            pltpu.make_async_copy(g_hbm.at[s1.at[j]], rows[b],
                                  gsems[b]).wait()
            pltpu.sync_copy(rows[b], acc.at[d1.at[j]], add=True)
            if j + 2 < KCH1:
                pltpu.async_copy(g_hbm.at[s1.at[j + 2]], rows[b], gsems[b])
        _edges_path(base1 + KCH1, KX1)

    plsc.subcore_barrier()

    @pl.when(s < NS - 1)
    def _():
        pltpu.sync_copy(acc.at[pl.ds(s * RPT, RPT)],
                        out_hbm.at[c, pl.ds(s * RPT, RPT)])

    @pl.when(s == NS - 1)
    def _():
        pltpu.sync_copy(acc.at[pl.ds((NS - 1) * RPT, RPT_LAST)],
                        out_hbm.at[c, pl.ds((NS - 1) * RPT, RPT_LAST)])


def _dis(dp0, dp1):
    deg = dp0[:, :1] + dp1[:, :1] + 1.0
    return lax.rsqrt(jnp.maximum(deg, 1.0))


def _mm_scale_body(x_ref, w_ref, dp0_ref, dp1_ref, o_ref):
    o_ref[...] = _dis(dp0_ref[0], dp1_ref[0]) * jnp.dot(
        x_ref[...], w_ref[...], preferred_element_type=jnp.float32)


def _mm_scale(x, w, degp):
    return pl.pallas_call(
        _mm_scale_body,
        grid=(N // RB,),
        in_specs=[pl.BlockSpec((RB, D), lambda i: (i, 0)),
                  pl.BlockSpec((D, D), lambda i: (0, 0)),
                  pl.BlockSpec((1, RB, D), lambda i: (0, i, 0)),
                  pl.BlockSpec((1, RB, D), lambda i: (1, i, 0))],
        out_specs=pl.BlockSpec((RB, D), lambda i: (i, 0)),
        out_shape=jax.ShapeDtypeStruct((N, D), jnp.float32),
    )(x, w, degp, degp)


def _layer_body(p0_ref, p1_ref, g_ref, dp0_ref, dp1_ref, b_ref, w_ref, o_ref):
    dis = _dis(dp0_ref[0], dp1_ref[0])
    h = jnp.maximum(
        dis * (p0_ref[0] + p1_ref[0] + g_ref[...]) + b_ref[...], 0.0)
    o_ref[...] = dis * jnp.dot(h, w_ref[...],
                               preferred_element_type=jnp.float32)


def _layer(p, g, degp, b, w):
    return pl.pallas_call(
        _layer_body,
        grid=(N // RB,),
        in_specs=[pl.BlockSpec((1, RB, D), lambda i: (0, i, 0)),
                  pl.BlockSpec((1, RB, D), lambda i: (1, i, 0)),
                  pl.BlockSpec((RB, D), lambda i: (i, 0)),
                  pl.BlockSpec((1, RB, D), lambda i: (0, i, 0)),
                  pl.BlockSpec((1, RB, D), lambda i: (1, i, 0)),
                  pl.BlockSpec((1, D), lambda i: (0, 0)),
                  pl.BlockSpec((D, D), lambda i: (0, 0))],
        out_specs=pl.BlockSpec((RB, D), lambda i: (i, 0)),
        out_shape=jax.ShapeDtypeStruct((N, D), jnp.float32),
    )(p, p, g, degp, degp, b, w)


def _final_body(p0_ref, p1_ref, g_ref, dp0_ref, dp1_ref, b_ref, bat_ref,
                l1w_ref, l1b_ref, l2w_ref, l2b_ref, o_ref, sums, cnts):
    i = pl.program_id(0)

    @pl.when(i == 0)
    def _():
        sums[...] = jnp.zeros_like(sums)
        cnts[...] = jnp.zeros_like(cnts)

    dis = _dis(dp0_ref[0], dp1_ref[0])
    h = jnp.maximum(
        dis * (p0_ref[0] + p1_ref[0] + g_ref[...]) + b_ref[...], 0.0)
    gid = lax.broadcasted_iota(jnp.int32, (RB, G), 1)
    mask = (bat_ref[...] == gid).astype(jnp.float32)
    dn = (((0,), (0,)), ((), ()))
    sums[...] += lax.dot_general(mask, h, dn,
                                 preferred_element_type=jnp.float32)
    cnts[...] += lax.dot_general(mask, jnp.ones((RB, D), jnp.float32), dn,
                                 preferred_element_type=jnp.float32)

    @pl.when(i == (N // RB) - 1)
    def _():
        pooled = sums[...] / jnp.maximum(cnts[...], 1.0)
        z = jnp.maximum(
            jnp.dot(pooled, l1w_ref[...],
                    preferred_element_type=jnp.float32) + l1b_ref[...], 0.0)
        o_ref[...] = jnp.dot(z, l2w_ref[...],
                             preferred_element_type=jnp.float32) + l2b_ref[...]


def _final(p, g, degp, b, bat, l1w, l1b, l2w, l2b):
    return pl.pallas_call(
        _final_body,
        grid=(N // RB,),
        in_specs=[pl.BlockSpec((1, RB, D), lambda i: (0, i, 0)),
                  pl.BlockSpec((1, RB, D), lambda i: (1, i, 0)),
                  pl.BlockSpec((RB, D), lambda i: (i, 0)),
                  pl.BlockSpec((1, RB, D), lambda i: (0, i, 0)),
                  pl.BlockSpec((1, RB, D), lambda i: (1, i, 0)),
                  pl.BlockSpec((1, D), lambda i: (0, 0)),
                  pl.BlockSpec((RB, 1), lambda i: (i, 0)),
                  pl.BlockSpec((D, D), lambda i: (0, 0)),
                  pl.BlockSpec((1, D), lambda i: (0, 0)),
                  pl.BlockSpec((D, C), lambda i: (0, 0)),
                  pl.BlockSpec((1, C), lambda i: (0, 0))],
        out_specs=pl.BlockSpec((G, C), lambda i: (0, 0)),
        out_shape=jax.ShapeDtypeStruct((G, C), jnp.float32),
        scratch_shapes=[pltpu.VMEM((G, D), jnp.float32),
                        pltpu.VMEM((G, D), jnp.float32)],
    )(p, p, g, degp, degp, b, bat, l1w, l1b, l2w, l2b)


def kernel(x, edge_index, batch, W1, b1, W2, b2, W3, b3,
           lin1_W, lin1_b, lin2_W, lin2_b):
    src = edge_index[0].astype(jnp.int32)
    dst = edge_index[1].astype(jnp.int32)
    pad = EPAD - E
    pidx = jnp.arange(pad, dtype=jnp.int32) % (NPAD - N)
    srcp = jnp.concatenate([src, pidx % CH]).reshape(TOTCH, CH)
    dstp = jnp.concatenate([dst, N + pidx]).reshape(TOTCH, CH)

    degp = _deg_kernel(dstp.reshape(NT, KCH, CH))

    g0 = _mm_scale(x, W1, degp)
    p = _scatter_kernel(g0, srcp, dstp)
    g1 = _layer(p, g0, degp, b1.reshape(1, D), W2)
    p = _scatter_kernel(g1, srcp, dstp)
    g2 = _layer(p, g1, degp, b2.reshape(1, D), W3)
    p = _scatter_kernel(g2, srcp, dstp)
    return _final(p, g2, degp, b3.reshape(1, D),
                  batch.reshape(N, 1).astype(jnp.int32),
                  lin1_W, lin1_b.reshape(1, D), lin2_W, lin2_b.reshape(1, C))


# speedup vs baseline: 1.2081x; 1.0474x over previous
import functools

import jax
import jax.numpy as jnp
from jax import lax
from jax.experimental import pallas as pl
from jax.experimental.pallas import tpu as pltpu
from jax.experimental.pallas import tpu_sc as plsc

N = 10000
D = 128
E = 320000
G = 64
C = 16

NC = 2
NS = 16
NT = NC * NS
CH = 128
KCH = 80
IB = 8
KCH0 = 88
KCH1 = 40
KX1 = 32
NBLK0 = KCH0 // IB
C1ROWS = NS * (KCH1 + KX1)
TOTCH = NS * (KCH0 + KCH1 + KX1)
EPAD = TOTCH * CH
NPAD = 10112
RPT = 624
RPT_LAST = N - RPT * (NS - 1)
ZCH = NPAD // NS // CH

RB = 1000

_mesh = plsc.VectorSubcoreMesh(core_axis_name="c", subcore_axis_name="s")


@functools.partial(
    pl.kernel,
    out_type=jax.ShapeDtypeStruct((NC, N, D), jnp.float32),
    mesh=_mesh,
    scratch_types=[
        pltpu.VMEM((KCH, CH), jnp.int32),
        pltpu.VMEM((CH, D), jnp.float32),
        pltpu.VMEM((CH, D), jnp.float32),
        pltpu.VMEM_SHARED((NPAD, D), jnp.float32),
    ],
)
def _deg_kernel(dst_hbm, out_hbm, didx, ones, zeros, acc):
    c = lax.axis_index("c")
    s = lax.axis_index("s")
    t = c * NS + s

    @pl.loop(0, CH)
    def _(i):
        @pl.loop(0, D, step=16)
        def _(jj):
            ones[i, pl.ds(jj, 16)] = jnp.ones((16,), jnp.float32)
            zeros[i, pl.ds(jj, 16)] = jnp.zeros((16,), jnp.float32)

    @pl.loop(0, NPAD // NS // CH)
    def _(k):
        pltpu.sync_copy(zeros, acc.at[pl.ds(s * (NPAD // NS) + k * CH, CH)])

    pltpu.sync_copy(zeros, acc.at[pl.ds(s * (NPAD // NS) + NPAD // NS - CH, CH)])

    plsc.subcore_barrier()

    pltpu.sync_copy(dst_hbm.at[t], didx)

    @pl.loop(0, KCH)
    def _(j):
        pltpu.sync_copy(ones, acc.at[didx.at[j]], add=True)

    plsc.subcore_barrier()

    @pl.when(s < NS - 1)
    def _():
        pltpu.sync_copy(acc.at[pl.ds(s * RPT, RPT)],
                        out_hbm.at[c, pl.ds(s * RPT, RPT)])

    @pl.when(s == NS - 1)
    def _():
        pltpu.sync_copy(acc.at[pl.ds((NS - 1) * RPT, RPT_LAST)],
                        out_hbm.at[c, pl.ds((NS - 1) * RPT, RPT_LAST)])


@functools.partial(
    pl.kernel,
    out_type=jax.ShapeDtypeStruct((NC, N, D), jnp.float32),
    mesh=_mesh,
    scratch_types=[
        pltpu.VMEM((IB, CH), jnp.int32),
        pltpu.VMEM((IB, CH), jnp.int32),
        pltpu.VMEM((IB, CH), jnp.int32),
        pltpu.VMEM((IB, CH), jnp.int32),
        pltpu.VMEM((IB, CH), jnp.int32),
        pltpu.VMEM((IB, CH), jnp.int32),
        pltpu.VMEM((KCH1, CH), jnp.int32),
        pltpu.VMEM((KCH1, CH), jnp.int32),
        pltpu.VMEM((CH, D), jnp.float32),
        pltpu.VMEM((CH, D), jnp.float32),
        pltpu.VMEM_SHARED((NPAD, D), jnp.float32),
        pltpu.SemaphoreType.DMA,
        pltpu.SemaphoreType.DMA,
        pltpu.SemaphoreType.DMA,
        pltpu.SemaphoreType.DMA,
        pltpu.SemaphoreType.DMA,
        pltpu.SemaphoreType.DMA,
        pltpu.SemaphoreType.DMA,
        pltpu.SemaphoreType.DMA,
    ],
)
def _scatter_kernel(g_hbm, src_hbm, dst_hbm, out_hbm,
                    sA, sB, sC, dA, dB, dC, s1, d1, rows0, rows1, acc,
                    gsem0, gsem1, ssemA, ssemB, ssemC, dsemA, dsemB, dsemC):
    c = lax.axis_index("c")
    s = lax.axis_index("s")
    rows = [rows0, rows1]
    gsems = [gsem0, gsem1]
    sbufs = [sA, sB, sC]
    ssems = [ssemA, ssemB, ssemC]
    dbufs = [dA, dB, dC]
    dsems = [dsemA, dsemB, dsemC]

    @pl.loop(0, CH)
    def _(i):
        @pl.loop(0, D, step=16)
        def _(jj):
            rows0[i, pl.ds(jj, 16)] = jnp.zeros((16,), jnp.float32)

    @pl.loop(0, NPAD // NS // CH)
    def _(k):
        pltpu.sync_copy(rows0, acc.at[pl.ds(s * (NPAD // NS) + k * CH, CH)])

    pltpu.sync_copy(rows0, acc.at[pl.ds(s * (NPAD // NS) + NPAD // NS - CH, CH)])

    plsc.subcore_barrier()

    def _guard(cond, fn):
        if isinstance(cond, bool):
            if cond:
                fn()
        else:
            pl.when(cond)(fn)

    def _edges_path(cb, kch):
        nblk = kch // IB

        def sload(blk, bi):
            pltpu.async_copy(src_hbm.at[pl.ds(cb + blk * IB, IB)],
                             sbufs[bi], ssems[bi])

        def swait(blk, bi):
            pltpu.make_async_copy(src_hbm.at[pl.ds(cb + blk * IB, IB)],
                                  sbufs[bi], ssems[bi]).wait()

        def dload(blk, bi):
            pltpu.async_copy(dst_hbm.at[pl.ds(cb + blk * IB, IB)],
                             dbufs[bi], dsems[bi])

        def dwait(blk, bi):
            pltpu.make_async_copy(dst_hbm.at[pl.ds(cb + blk * IB, IB)],
                                  dbufs[bi], dsems[bi]).wait()

        def do_block(blk, bi):
            cur_s, next_s = sbufs[bi], sbufs[(bi + 1) % 3]
            cur_d = dbufs[bi]

            def _wait_next(blk=blk, bi=bi):
                swait(blk + 1, (bi + 1) % 3)
                dwait(blk + 1, (bi + 1) % 3)

            def _load_next2(blk=blk, bi=bi):
                sload(blk + 2, (bi + 2) % 3)
                dload(blk + 2, (bi + 2) % 3)

            _guard(blk + 1 < nblk, _wait_next)
            _guard(blk + 2 < nblk, _load_next2)

            @pl.loop(0, (IB - 2) // 2)
            def _(jj, cur_s=cur_s, cur_d=cur_d):
                for b in range(2):
                    i = jj * 2 + b
                    pltpu.make_async_copy(g_hbm.at[cur_s.at[i]], rows[b],
                                          gsems[b]).wait()
                    pltpu.sync_copy(rows[b], acc.at[cur_d.at[i]], add=True)
                    pltpu.async_copy(g_hbm.at[cur_s.at[i + 2]], rows[b],
                                     gsems[b])

            for i in (IB - 2, IB - 1):
                b = i % 2
                pltpu.make_async_copy(g_hbm.at[cur_s.at[i]], rows[b],
                                      gsems[b]).wait()
                pltpu.sync_copy(rows[b], acc.at[cur_d.at[i]], add=True)
                def _refire(i=i, b=b, next_s=next_s):
                    pltpu.async_copy(g_hbm.at[next_s.at[i + 2 - IB]],
                                     rows[b], gsems[b])

                _guard(blk * IB + i + 2 < kch, _refire)

        sload(0, 0)
        sload(1, 1)
        dload(0, 0)
        dload(1, 1)
        swait(0, 0)
        pltpu.async_copy(g_hbm.at[sbufs[0].at[0]], rows0, gsem0)
        pltpu.async_copy(g_hbm.at[sbufs[0].at[1]], rows1, gsem1)
        dwait(0, 0)

        @pl.loop(0, nblk // 3)
        def _(gg):
            for bi in range(3):
                do_block(gg * 3 + bi, bi)

        for l in range(nblk % 3):
            blk = nblk - (nblk % 3) + l
            do_block(blk, blk % 3)

    @pl.when(c == 0)
    def _():
        _edges_path(C1ROWS + s * KCH0, KCH0)

    @pl.when(c == 1)
    def _():
        base1 = s * (KCH1 + KX1)
        pltpu.sync_copy(src_hbm.at[pl.ds(base1, KCH1)], s1)
        pltpu.sync_copy(dst_hbm.at[pl.ds(base1, KCH1)], d1)
        pltpu.async_copy(g_hbm.at[s1.at[0]], rows0, gsem0)
        pltpu.async_copy(g_hbm.at[s1.at[1]], rows1, gsem1)
        for j in range(KCH1):
            b = j % 2
            pltpu.make_async_copy(g_hbm.at[s1.at[j]], rows[b],
                                  gsems[b]).wait()
            pltpu.sync_copy(rows[b], acc.at[d1.at[j]], add=True)
            if j + 2 < KCH1:
                pltpu.async_copy(g_hbm.at[s1.at[j + 2]], rows[b], gsems[b])
        _edges_path(base1 + KCH1, KX1)

    plsc.subcore_barrier()

    @pl.when(s < NS - 1)
    def _():
        pltpu.sync_copy(acc.at[pl.ds(s * RPT, RPT)],
                        out_hbm.at[c, pl.ds(s * RPT, RPT)])

    @pl.when(s == NS - 1)
    def _():
        pltpu.sync_copy(acc.at[pl.ds((NS - 1) * RPT, RPT_LAST)],
                        out_hbm.at[c, pl.ds((NS - 1) * RPT, RPT_LAST)])


def _dis(dp0, dp1):
    deg = dp0[:, :1] + dp1[:, :1] + 1.0
    return lax.rsqrt(jnp.maximum(deg, 1.0))


def _mm_scale_body(x_ref, w_ref, dp0_ref, dp1_ref, o_ref):
    o_ref[...] = _dis(dp0_ref[0], dp1_ref[0]) * jnp.dot(
        x_ref[...], w_ref[...], preferred_element_type=jnp.float32)


def _mm_scale(x, w, degp):
    return pl.pallas_call(
        _mm_scale_body,
        grid=(N // RB,),
        in_specs=[pl.BlockSpec((RB, D), lambda i: (i, 0)),
                  pl.BlockSpec((D, D), lambda i: (0, 0)),
                  pl.BlockSpec((1, RB, D), lambda i: (0, i, 0)),
                  pl.BlockSpec((1, RB, D), lambda i: (1, i, 0))],
        out_specs=pl.BlockSpec((RB, D), lambda i: (i, 0)),
        out_shape=jax.ShapeDtypeStruct((N, D), jnp.float32),
    )(x, w, degp, degp)


def _layer_body(p0_ref, p1_ref, g_ref, dp0_ref, dp1_ref, b_ref, w_ref, o_ref):
    dis = _dis(dp0_ref[0], dp1_ref[0])
    h = jnp.maximum(
        dis * (p0_ref[0] + p1_ref[0] + g_ref[...]) + b_ref[...], 0.0)
    o_ref[...] = dis * jnp.dot(h, w_ref[...],
                               preferred_element_type=jnp.float32)


def _layer(p, g, degp, b, w):
    return pl.pallas_call(
        _layer_body,
        grid=(N // RB,),
        in_specs=[pl.BlockSpec((1, RB, D), lambda i: (0, i, 0)),
                  pl.BlockSpec((1, RB, D), lambda i: (1, i, 0)),
                  pl.BlockSpec((RB, D), lambda i: (i, 0)),
                  pl.BlockSpec((1, RB, D), lambda i: (0, i, 0)),
                  pl.BlockSpec((1, RB, D), lambda i: (1, i, 0)),
                  pl.BlockSpec((1, D), lambda i: (0, 0)),
                  pl.BlockSpec((D, D), lambda i: (0, 0))],
        out_specs=pl.BlockSpec((RB, D), lambda i: (i, 0)),
        out_shape=jax.ShapeDtypeStruct((N, D), jnp.float32),
    )(p, p, g, degp, degp, b, w)


def _final_body(p0_ref, p1_ref, g_ref, dp0_ref, dp1_ref, b_ref, bat_ref,
                l1w_ref, l1b_ref, l2w_ref, l2b_ref, o_ref, sums, cnts):
    i = pl.program_id(0)

    @pl.when(i == 0)
    def _():
        sums[...] = jnp.zeros_like(sums)
        cnts[...] = jnp.zeros_like(cnts)

    dis = _dis(dp0_ref[0], dp1_ref[0])
    h = jnp.maximum(
        dis * (p0_ref[0] + p1_ref[0] + g_ref[...]) + b_ref[...], 0.0)
    gid = lax.broadcasted_iota(jnp.int32, (RB, G), 1)
    mask = (bat_ref[...] == gid).astype(jnp.float32)
    dn = (((0,), (0,)), ((), ()))
    sums[...] += lax.dot_general(mask, h, dn,
                                 preferred_element_type=jnp.float32)
    cnts[...] += lax.dot_general(mask, jnp.ones((RB, D), jnp.float32), dn,
                                 preferred_element_type=jnp.float32)

    @pl.when(i == (N // RB) - 1)
    def _():
        pooled = sums[...] / jnp.maximum(cnts[...], 1.0)
        z = jnp.maximum(
            jnp.dot(pooled, l1w_ref[...],
                    preferred_element_type=jnp.float32) + l1b_ref[...], 0.0)
        o_ref[...] = jnp.dot(z, l2w_ref[...],
                             preferred_element_type=jnp.float32) + l2b_ref[...]


def _final(p, g, degp, b, bat, l1w, l1b, l2w, l2b):
    return pl.pallas_call(
        _final_body,
        grid=(N // RB,),
        in_specs=[pl.BlockSpec((1, RB, D), lambda i: (0, i, 0)),
                  pl.BlockSpec((1, RB, D), lambda i: (1, i, 0)),
                  pl.BlockSpec((RB, D), lambda i: (i, 0)),
                  pl.BlockSpec((1, RB, D), lambda i: (0, i, 0)),
                  pl.BlockSpec((1, RB, D), lambda i: (1, i, 0)),
                  pl.BlockSpec((1, D), lambda i: (0, 0)),
                  pl.BlockSpec((RB, 1), lambda i: (i, 0)),
                  pl.BlockSpec((D, D), lambda i: (0, 0)),
                  pl.BlockSpec((1, D), lambda i: (0, 0)),
                  pl.BlockSpec((D, C), lambda i: (0, 0)),
                  pl.BlockSpec((1, C), lambda i: (0, 0))],
        out_specs=pl.BlockSpec((G, C), lambda i: (0, 0)),
        out_shape=jax.ShapeDtypeStruct((G, C), jnp.float32),
        scratch_shapes=[pltpu.VMEM((G, D), jnp.float32),
                        pltpu.VMEM((G, D), jnp.float32)],
    )(p, p, g, degp, degp, b, bat, l1w, l1b, l2w, l2b)


def kernel(x, edge_index, batch, W1, b1, W2, b2, W3, b3,
           lin1_W, lin1_b, lin2_W, lin2_b):
    src = edge_index[0].astype(jnp.int32)
    dst = edge_index[1].astype(jnp.int32)
    pad = EPAD - E
    pidx = jnp.arange(pad, dtype=jnp.int32) % (NPAD - N)
    srcp = jnp.concatenate([src, pidx % CH]).reshape(TOTCH, CH)
    dstp = jnp.concatenate([dst, N + pidx]).reshape(TOTCH, CH)

    degp = _deg_kernel(dstp.reshape(NT, KCH, CH))

    g0 = _mm_scale(x, W1, degp)
    p = _scatter_kernel(g0, srcp, dstp)
    g1 = _layer(p, g0, degp, b1.reshape(1, D), W2)
    p = _scatter_kernel(g1, srcp, dstp)
    g2 = _layer(p, g1, degp, b2.reshape(1, D), W3)
    p = _scatter_kernel(g2, srcp, dstp)
    return _final(p, g2, degp, b3.reshape(1, D),
                  batch.reshape(N, 1).astype(jnp.int32),
                  lin1_W, lin1_b.reshape(1, D), lin2_W, lin2_b.reshape(1, C))
